# Initial kernel scaffold; baseline (speedup 1.0000x reference)
import jax
import jax.numpy as jnp
from jax.experimental import pallas as pl


def kernel(edge_index, feats_node, feats_edge, charges_init, w0_src, w0_ni, w0_nj, w0_fij, a0, b0, w1_src, w1_ni, w1_nj, w1_fij, a1, b1, w2_src, w2_ni, w2_nj, w2_fij, a2, b2, mlp_w0, mlp_b0, mlp_w1, mlp_b1):
    raise NotImplementedError("write your pallas kernel here")


# speedup vs baseline: 1.0001x; 1.0001x over previous
import jax
import jax.numpy as jnp
from jax.experimental import pallas as pl

N = 10000
E = 320000
HN = 64
HE = 64
H = 1


def _edge_softmax(e, dst, n):
    m = jax.ops.segment_max(e, dst, num_segments=n)
    m = jnp.where(jnp.isfinite(m), m, 0.0)
    ex = jnp.exp(e - m[dst])
    s = jax.ops.segment_sum(ex, dst, num_segments=n)
    return ex / s[dst]


def _egat(h, f, src, dst, Wsrc, Wni, Wnj, Wfij, attn, bias):
    n = h.shape[0]
    f_ni = h @ Wni.T
    f_nj = h @ Wnj.T
    f_fij = f @ Wfij.T
    f_out = f_ni[src] + f_nj[dst] + f_fij + bias
    f_out = jax.nn.leaky_relu(f_out, negative_slope=0.01)
    f_out = f_out.reshape(-1, H, HE)
    e = (f_out * attn).sum(axis=-1)
    a = _edge_softmax(e, dst, n)
    h_src = (h @ Wsrc.T).reshape(-1, H, HN)
    msg = h_src[src] * a[:, :, None]
    h_out = jax.ops.segment_sum(msg, dst, num_segments=n)
    return h_out, f_out


def _combine_body(d_ref, dr_ref, c_ref, o_ref):
    o_ref[...] = d_ref[...] + dr_ref[...] + c_ref[...]


def kernel(edge_index, feats_node, feats_edge, charges_init, w0_src, w0_ni, w0_nj, w0_fij, a0, b0, w1_src, w1_ni, w1_nj, w1_fij, a1, b1, w2_src, w2_ni, w2_nj, w2_fij, a2, b2, mlp_w0, mlp_b0, mlp_w1, mlp_b1):
    src = edge_index[0]
    dst = edge_index[1]
    n = feats_node.shape[0]
    h, f = feats_node, feats_edge
    layers = [(w0_src, w0_ni, w0_nj, w0_fij, a0, b0),
              (w1_src, w1_ni, w1_nj, w1_fij, a1, b1),
              (w2_src, w2_ni, w2_nj, w2_fij, a2, b2)]
    for (Ws, Wni, Wnj, Wf, at, bi) in layers:
        h, f = _egat(h, f, src, dst, Ws, Wni, Wnj, Wf, at, bi)
        h = jax.nn.relu(h).reshape(-1, HN * H)
        f = jax.nn.relu(f).reshape(-1, HE * H)
    x = jax.nn.selu(f @ mlp_w0.T + mlp_b0)
    inc = (x @ mlp_w1.T + mlp_b1).reshape(-1)
    delta = jax.ops.segment_sum(inc, dst, num_segments=n)
    delta_rev = jax.ops.segment_sum(-inc, src, num_segments=n)
    out = pl.pallas_call(
        _combine_body,
        out_shape=jax.ShapeDtypeStruct((n,), jnp.float32),
    )(delta, delta_rev, charges_init)
    return out


# speedup vs baseline: 3.3541x; 3.3539x over previous
import functools

import jax
import jax.numpy as jnp
from jax import lax
from jax.experimental import pallas as pl
from jax.experimental.pallas import tpu as pltpu
from jax.experimental.pallas import tpu_sc as plsc

N = 10000
E = 320000
HN = 64
HE = 64
H = 1

NC = 2
NS = 16
NW = NC * NS
L = 16

NP = 10240
B = 80
E2 = 327680
EPAD = E2 - E
ROWS2 = E2 // B
RW = ROWS2 // NW
EW2 = RW * B
RS = ROWS2 // NS
SCE = 2560
NCH = RS * B // SCE
NT = NP // NS

_SELU_A = 1.6732632423543772
_SELU_S = 1.0507009873554805

f32 = jnp.float32
i32 = jnp.int32


def _proj_body(*refs, combine):
    if combine:
        h0, h1, wni, wnj, wsrc, o_ni, o_nj, o_src = refs
        h = jnp.maximum(h0[...] + h1[...], 0.0)
    else:
        h_ref, wni, wnj, wsrc, o_ni, o_nj, o_src = refs
        h = h_ref[...]
    o_ni[...] = jnp.dot(h, wni[...], preferred_element_type=f32)
    o_nj[...] = jnp.dot(h, wnj[...], preferred_element_type=f32)
    o_src[...] = jnp.dot(h, wsrc[...], preferred_element_type=f32)


def _projections(h_or_parts, wniT, wnjT, wsrcT, combine):
    d = wniT.shape[0]
    nb = 10
    bn = N // nb
    hspec = pl.BlockSpec((bn, d), lambda i: (i, 0))
    wspec = pl.BlockSpec((d, HN), lambda i: (0, 0))
    ospec = pl.BlockSpec((bn, HN), lambda i: (i, 0))
    out = jax.ShapeDtypeStruct((N, HN), f32)
    if combine:
        in_specs = [hspec, hspec, wspec, wspec, wspec]
        args = (*h_or_parts, wniT, wnjT, wsrcT)
    else:
        in_specs = [hspec, wspec, wspec, wspec]
        args = (h_or_parts, wniT, wnjT, wsrcT)
    return pl.pallas_call(
        functools.partial(_proj_body, combine=combine),
        grid=(nb,),
        in_specs=in_specs,
        out_specs=[ospec, ospec, ospec],
        out_shape=[out, out, out],
    )(*args)


def _edge_mm_body(f_ref, w_ref, b_ref, o_ref, *, relu):
    f = f_ref[...]
    if relu:
        f = jnp.maximum(f, 0.0)
    o_ref[...] = jnp.dot(f, w_ref[...], preferred_element_type=f32) + b_ref[...]


def _edge_matmul(f, wT, bias, relu):
    d = wT.shape[0]
    nb = 100
    be = E // nb
    return pl.pallas_call(
        functools.partial(_edge_mm_body, relu=relu),
        grid=(nb,),
        in_specs=[pl.BlockSpec((be, d), lambda i: (i, 0)),
                  pl.BlockSpec((d, HE), lambda i: (0, 0)),
                  pl.BlockSpec((1, HE), lambda i: (0, 0))],
        out_specs=pl.BlockSpec((be, HE), lambda i: (i, 0)),
        out_shape=jax.ShapeDtypeStruct((E, HE), f32),
    )(f, wT, bias)


def _mlp_body(f_ref, w0_ref, b0_ref, w1_ref, b1_ref, o_ref):
    f = jnp.maximum(f_ref[...], 0.0)
    x = jnp.dot(f, w0_ref[...], preferred_element_type=f32) + b0_ref[...]
    x = _SELU_S * jnp.where(x > 0.0, x, _SELU_A * (jnp.exp(x) - 1.0))
    o_ref[...] = jnp.sum(x * w1_ref[...], axis=1) + b1_ref[0, 0]


def _final_mlp(fout, w0T, b0, w1, b1):
    nb = 625
    be = E // nb
    dm = w0T.shape[1]
    return pl.pallas_call(
        _mlp_body,
        grid=(nb,),
        in_specs=[pl.BlockSpec((be, HE), lambda i: (i, 0)),
                  pl.BlockSpec((HE, dm), lambda i: (0, 0)),
                  pl.BlockSpec((1, dm), lambda i: (0, 0)),
                  pl.BlockSpec((1, dm), lambda i: (0, 0)),
                  pl.BlockSpec((1, 1), lambda i: (0, 0))],
        out_specs=pl.BlockSpec((be,), lambda i: (i,)),
        out_shape=jax.ShapeDtypeStruct((E,), f32),
    )(fout, w0T, b0, w1, b1)


def _combine_body(p_ref, c_ref, o_ref):
    o_ref[...] = jnp.sum(p_ref[...], axis=0) + c_ref[...]


def _final_combine(parts, charges):
    return pl.pallas_call(
        _combine_body,
        out_shape=jax.ShapeDtypeStruct((N,), f32),
    )(parts, charges)


def _lane():
    return lax.iota(i32, L)


def _any_lanes(pend):
    cnt = plsc.all_reduce_population_count(pend != 0)
    return cnt[0] > 0


def _scatter_update(acc_ref, tag_ref, idx, val, op):
    tslot = lax.bitwise_and(idx, 1023)
    lane = _lane()

    def body(pend_i):
        pend = pend_i != 0
        plsc.store_scatter(tag_ref, [tslot], lane, mask=pend)
        got = plsc.load_gather(tag_ref, [tslot])
        win = jnp.logical_and(got == lane, pend)
        cur = plsc.load_gather(acc_ref, [idx])
        new = op(cur, val)
        plsc.store_scatter(acc_ref, [idx], new, mask=win)
        return jnp.where(win, 0, pend_i)

    lax.while_loop(_any_lanes, body, jnp.ones((L,), i32))


def _fill_1d(ref, n, value):
    def body(i, _):
        ref[pl.ds(i * L, L)] = jnp.full((L,), value, f32)
        return 0
    lax.fori_loop(0, n // L, body, 0)


def _fill_2d(ref, rows, cols, value):
    def body(i, _):
        r = i // (cols // L)
        k = lax.rem(i, cols // L)
        ref[r, pl.ds(k * L, L)] = jnp.full((L,), value, f32)
        return 0
    lax.fori_loop(0, rows * (cols // L), body, 0)


def _m8(x):
    return pl.multiple_of(x, 8)


def _edge_feat_kernel(src2, dst2, pni, pnj, fij, attnb):
    mesh = plsc.VectorSubcoreMesh(core_axis_name="c", subcore_axis_name="s")

    @functools.partial(
        pl.kernel,
        out_type=[jax.ShapeDtypeStruct((E, HE), f32),
                  jax.ShapeDtypeStruct((ROWS2, B), f32)],
        mesh=mesh,
        compiler_params=pltpu.CompilerParams(needs_layout_passes=False, use_tc_tiling_on_sc=False),
        scratch_types=[
            pltpu.VMEM((RW, B), i32),
            pltpu.VMEM((RW, B), i32),
            pltpu.VMEM((B, HE), f32),
            pltpu.VMEM((B, HE), f32),
            pltpu.VMEM((B, HE), f32),
            pltpu.VMEM((B, HE), f32),
            pltpu.VMEM((B,), f32),
            pltpu.VMEM((HE, L), f32),
        ],
    )
    def body(src_h, dst_h, pni_h, pnj_h, fij_h, attnb_h, fout_h, e_h,
             src_v, dst_v, ni_v, nj_v, fij_v, fo_v, e_v, attn_v):
        cid = lax.axis_index("c")
        sid = lax.axis_index("s")
        wid = cid * NS + sid
        row0 = _m8(wid * RW)
        ebase = wid * EW2

        pltpu.sync_copy(src_h.at[pl.ds(row0, RW), :], src_v)
        pltpu.sync_copy(dst_h.at[pl.ds(row0, RW), :], dst_v)
        pltpu.sync_copy(attnb_h, attn_v)

        def chunk(cc, _):
            e0 = _m8(ebase + cc * B)

            @pl.when(e0 < E)
            def _():
                pltpu.sync_copy(pni_h.at[src_v.at[cc]], ni_v)
                pltpu.sync_copy(pnj_h.at[dst_v.at[cc]], nj_v)
                pltpu.sync_copy(fij_h.at[pl.ds(e0, B), :], fij_v)

                def ew(i, _):
                    r = i // 4
                    k = lax.rem(i, 4)
                    sl = pl.ds(k * L, L)
                    x = ni_v[r, sl] + nj_v[r, sl] + fij_v[r, sl]
                    fo_v[r, sl] = jnp.maximum(x, 0.0) + 0.01 * jnp.minimum(x, 0.0)
                    return 0
                lax.fori_loop(0, B * 4, ew, 0)

                def egroup(g, _):
                    rows = _lane() + g * L

                    def edim(d, acc):
                        v = plsc.load_gather(fo_v, [rows, jnp.full((L,), d, i32)])
                        return acc + v * attn_v[d, :]
                    acc = lax.fori_loop(0, HE, edim, jnp.zeros((L,), f32))
                    e_v[pl.ds(g * L, L)] = acc
                    return 0
                lax.fori_loop(0, B // L, egroup, 0)

                pltpu.sync_copy(fo_v, fout_h.at[pl.ds(e0, B), :])
                pltpu.sync_copy(e_v, e_h.at[row0 + cc])

            @pl.when(e0 >= E)
            def _():
                def zf(g, _):
                    e_v[pl.ds(g * L, L)] = jnp.zeros((L,), f32)
                    return 0
                lax.fori_loop(0, B // L, zf, 0)
                pltpu.sync_copy(e_v, e_h.at[row0 + cc])

            return 0

        lax.fori_loop(0, RW, chunk, 0)

    return body(src2, dst2, pni, pnj, fij, attnb)


def _softmax_message_kernel(e2, src2, dst2, psrc):
    mesh = plsc.VectorSubcoreMesh(core_axis_name="c", subcore_axis_name="s")

    @functools.partial(
        pl.kernel,
        out_type=[jax.ShapeDtypeStruct((NC, N, HN), f32),
                  jax.ShapeDtypeStruct((NC, NS, NP), f32),
                  jax.ShapeDtypeStruct((NC, 2, NP), f32)],
        mesh=mesh,
        compiler_params=pltpu.CompilerParams(needs_layout_passes=False, use_tc_tiling_on_sc=False),
        scratch_types=[
            pltpu.VMEM((NP,), f32),
            pltpu.VMEM((NP,), f32),
            pltpu.VMEM((NP,), f32),
            pltpu.VMEM((NP,), f32),
            pltpu.VMEM((1024,), i32),
            pltpu.VMEM((SCE // B, B), f32),
            pltpu.VMEM((SCE // B, B), i32),
            pltpu.VMEM((RW, B), i32),
            pltpu.VMEM((RW, B), i32),
            pltpu.VMEM((RW, B), f32),
            pltpu.VMEM((B, HN), f32),
            pltpu.VMEM((B,), f32),
            pltpu.VMEM((NT,), f32),
            pltpu.VMEM((NT,), f32),
            pltpu.VMEM((B, HN), f32),
            pltpu.VMEM_SHARED((NP, HN), f32),
        ],
    )
    def body(e_h, src_h, dst_h, psrc_h, out_h, stat_h, fin_h,
             m_loc, s_loc, m_t, s_t, tag_v, ch_e, ch_d,
             msg_s2, msg_d2, msg_e, rows_v, a_v, cmb_a, cmb_i, zb,
             h_acc):
        cid = lax.axis_index("c")
        sid = lax.axis_index("s")
        wid = cid * NS + sid
        row0 = _m8(wid * RW)
        ebase = wid * EW2

        _fill_1d(m_loc, NP, -3.0e38)
        _fill_2d(zb, B, HN, 0.0)

        def hz(j, _):
            r0 = _m8(sid * NT + j * B)
            pltpu.sync_copy(zb, h_acc.at[pl.ds(r0, B), :])
            return 0
        lax.fori_loop(0, NT // B, hz, 0)

        SR = SCE // B

        def max_chunk(cc, _):
            o = _m8(sid * RS + cc * SR)
            pltpu.sync_copy(e_h.at[pl.ds(o, SR), :], ch_e)
            pltpu.sync_copy(dst_h.at[pl.ds(o, SR), :], ch_d)

            def vreg(i, _):
                r = i // (B // L)
                sl = pl.ds(lax.rem(i, B // L) * L, L)
                _scatter_update(m_loc, tag_v, ch_d[r, sl], ch_e[r, sl],
                                jnp.maximum)
                return 0
            lax.fori_loop(0, SCE // L, vreg, 0)
            return 0
        lax.fori_loop(0, NCH, max_chunk, 0)

        pltpu.sync_copy(m_loc, stat_h.at[cid, sid])
        plsc.subcore_barrier()

        col0 = _m8(sid * NT)
        pltpu.sync_copy(stat_h.at[cid, 0, pl.ds(col0, NT)], cmb_a)

        def mrow(t, _):
            pltpu.sync_copy(stat_h.at[cid, t, pl.ds(col0, NT)], cmb_i)

            def mv(i, _):
                sl = pl.ds(i * L, L)
                cmb_a[sl] = jnp.maximum(cmb_a[sl], cmb_i[sl])
                return 0
            lax.fori_loop(0, NT // L, mv, 0)
            return 0
        lax.fori_loop(1, NS, mrow, 0)
        pltpu.sync_copy(cmb_a, fin_h.at[cid, 0, pl.ds(col0, NT)])
        plsc.subcore_barrier()
        pltpu.sync_copy(fin_h.at[cid, 0], m_t)

        _fill_1d(s_loc, NP, 0.0)

        def sum_chunk(cc, _):
            o = _m8(sid * RS + cc * SR)
            pltpu.sync_copy(e_h.at[pl.ds(o, SR), :], ch_e)
            pltpu.sync_copy(dst_h.at[pl.ds(o, SR), :], ch_d)

            def vreg(i, _):
                r = i // (B // L)
                sl = pl.ds(lax.rem(i, B // L) * L, L)
                dv = ch_d[r, sl]
                ex = jnp.exp(ch_e[r, sl] - plsc.load_gather(m_t, [dv]))
                _scatter_update(s_loc, tag_v, dv, ex, jnp.add)
                return 0
            lax.fori_loop(0, SCE // L, vreg, 0)
            return 0
        lax.fori_loop(0, NCH, sum_chunk, 0)

        pltpu.sync_copy(s_loc, stat_h.at[cid, sid])
        plsc.subcore_barrier()

        pltpu.sync_copy(stat_h.at[cid, 0, pl.ds(col0, NT)], cmb_a)

        def srow(t, _):
            pltpu.sync_copy(stat_h.at[cid, t, pl.ds(col0, NT)], cmb_i)

            def sv(i, _):
                sl = pl.ds(i * L, L)
                cmb_a[sl] = cmb_a[sl] + cmb_i[sl]
                return 0
            lax.fori_loop(0, NT // L, sv, 0)
            return 0
        lax.fori_loop(1, NS, srow, 0)
        pltpu.sync_copy(cmb_a, fin_h.at[cid, 1, pl.ds(col0, NT)])
        plsc.subcore_barrier()
        pltpu.sync_copy(fin_h.at[cid, 1], s_t)

        pltpu.sync_copy(src_h.at[pl.ds(row0, RW), :], msg_s2)
        pltpu.sync_copy(dst_h.at[pl.ds(row0, RW), :], msg_d2)
        pltpu.sync_copy(e_h.at[pl.ds(row0, RW), :], msg_e)

        def chunk(cc, _):
            e0 = ebase + cc * B

            @pl.when(e0 < E)
            def _():
                pltpu.sync_copy(psrc_h.at[msg_s2.at[cc]], rows_v)

                def agroup(g, _):
                    sl = pl.ds(g * L, L)
                    dv = msg_d2[cc, sl]
                    ex = jnp.exp(msg_e[cc, sl] - plsc.load_gather(m_t, [dv]))
                    a_v[sl] = ex / plsc.load_gather(s_t, [dv])
                    return 0
                lax.fori_loop(0, B // L, agroup, 0)

                def scale(r, _):
                    ab = plsc.load_gather(a_v, [jnp.full((L,), r, i32)])
                    for k in range(HN // L):
                        sl = pl.ds(k * L, L)
                        rows_v[r, sl] = rows_v[r, sl] * ab
                    return 0
                lax.fori_loop(0, B, scale, 0)

                pltpu.sync_copy(rows_v, h_acc.at[msg_d2.at[cc]], add=True)

            return 0
        lax.fori_loop(0, RW, chunk, 0)

        plsc.subcore_barrier()

        def hout(j, _):
            r0 = _m8(sid * NT + j * B)

            @pl.when(r0 < N)
            def _():
                pltpu.sync_copy(h_acc.at[pl.ds(r0, B), :],
                                out_h.at[cid, pl.ds(r0, B), :])
            return 0
        lax.fori_loop(0, NT // B, hout, 0)

    return body(e2, src2, dst2, psrc)


def _delta_kernel(inc1, src1, dst1):
    mesh = plsc.VectorSubcoreMesh(core_axis_name="c", subcore_axis_name="s")

    @functools.partial(
        pl.kernel,
        out_type=jax.ShapeDtypeStruct((NW * N,), f32),
        mesh=mesh,
        compiler_params=pltpu.CompilerParams(needs_layout_passes=False, use_tc_tiling_on_sc=False),
        scratch_types=[
            pltpu.VMEM((NP,), f32),
            pltpu.VMEM((1024,), i32),
            pltpu.VMEM((SCE,), f32),
            pltpu.VMEM((SCE,), i32),
            pltpu.VMEM((SCE,), i32),
        ],
    )
    def body(inc_h, src_h, dst_h, out_h, d_loc, tag_v, ch_i, ch_s, ch_d):
        cid = lax.axis_index("c")
        sid = lax.axis_index("s")
        wid = cid * NS + sid
        ebase = wid * EW2

        _fill_1d(d_loc, NP, 0.0)

        def chunk(cc, _):
            o = _m8(ebase + cc * SCE)
            pltpu.sync_copy(inc_h.at[pl.ds(o, SCE)], ch_i)
            pltpu.sync_copy(src_h.at[pl.ds(o, SCE)], ch_s)
            pltpu.sync_copy(dst_h.at[pl.ds(o, SCE)], ch_d)

            def vreg(i, _):
                sl = pl.ds(i * L, L)
                iv = ch_i[sl]
                _scatter_update(d_loc, tag_v, ch_d[sl], iv, jnp.add)
                _scatter_update(d_loc, tag_v, ch_s[sl], -iv, jnp.add)
                return 0
            lax.fori_loop(0, SCE // L, vreg, 0)
            return 0
        lax.fori_loop(0, EW2 // SCE, chunk, 0)

        pltpu.sync_copy(d_loc.at[pl.ds(0, N)], out_h.at[pl.ds(_m8(wid * N), N)])

    return body(inc1, src1, dst1)


def kernel(edge_index, feats_node, feats_edge, charges_init,
           w0_src, w0_ni, w0_nj, w0_fij, a0, b0,
           w1_src, w1_ni, w1_nj, w1_fij, a1, b1,
           w2_src, w2_ni, w2_nj, w2_fij, a2, b2,
           mlp_w0, mlp_b0, mlp_w1, mlp_b1):
    src = edge_index[0]
    dst = edge_index[1]
    pad0 = jnp.zeros((EPAD,), i32)
    dummy = (N + (jnp.arange(EPAD, dtype=i32) % (NP - N))).astype(i32)
    src2 = jnp.concatenate([src, pad0]).reshape(ROWS2, B)
    src1 = jnp.concatenate([src, dummy])
    dst1 = jnp.concatenate([dst, dummy])
    dst2 = dst1.reshape(ROWS2, B)

    layers = [(w0_src, w0_ni, w0_nj, w0_fij, a0, b0),
              (w1_src, w1_ni, w1_nj, w1_fij, a1, b1),
              (w2_src, w2_ni, w2_nj, w2_fij, a2, b2)]

    h_parts = None
    fout = None
    for li, (Ws, Wni, Wnj, Wf, at, bi) in enumerate(layers):
        attnb = jnp.broadcast_to(at.reshape(HE, 1), (HE, L))
        if li == 0:
            pni, pnj, psrc = _projections(feats_node, Wni.T, Wnj.T, Ws.T,
                                          combine=False)
            fij = _edge_matmul(feats_edge, Wf.T, bi.reshape(1, HE), relu=False)
        else:
            pni, pnj, psrc = _projections(h_parts, Wni.T, Wnj.T, Ws.T,
                                          combine=True)
            fij = _edge_matmul(fout, Wf.T, bi.reshape(1, HE), relu=True)
        fout, e2 = _edge_feat_kernel(src2, dst2, pni, pnj, fij, attnb)
        hp, _, _ = _softmax_message_kernel(e2, src2, dst2, psrc)
        h_parts = (hp[0], hp[1])

    inc = _final_mlp(fout, mlp_w0.T, mlp_b0.reshape(1, -1),
                     mlp_w1.reshape(1, -1), mlp_b1.reshape(1, 1))
    inc1 = jnp.concatenate([inc, jnp.zeros((EPAD,), f32)])
    parts = _delta_kernel(inc1, src1, dst1)
    return _final_combine(parts.reshape(NW, N), charges_init)


# speedup vs baseline: 4.2901x; 1.2791x over previous
import functools

import jax
import jax.numpy as jnp
from jax import lax
from jax.experimental import pallas as pl
from jax.experimental.pallas import tpu as pltpu
from jax.experimental.pallas import tpu_sc as plsc

N = 10000
E = 320000
HN = 64
HE = 64
H = 1

NC = 2
NS = 16
NW = NC * NS
L = 16

NP = 10240
B = 80
E2 = 327680
EPAD = E2 - E
ROWS2 = E2 // B
RW = ROWS2 // NW
EW2 = RW * B
RS = ROWS2 // NS
SCE = 2560
NCH = RS * B // SCE
NT = NP // NS

_SELU_A = 1.6732632423543772
_SELU_S = 1.0507009873554805

f32 = jnp.float32
i32 = jnp.int32


def _proj_body(*refs, combine):
    if combine:
        h0, h1, wni, wnj, wsrc, o_ni, o_nj, o_src = refs
        h = jnp.maximum(h0[...] + h1[...], 0.0)
    else:
        h_ref, wni, wnj, wsrc, o_ni, o_nj, o_src = refs
        h = h_ref[...]
    o_ni[...] = jnp.dot(h, wni[...], preferred_element_type=f32)
    o_nj[...] = jnp.dot(h, wnj[...], preferred_element_type=f32)
    o_src[...] = jnp.dot(h, wsrc[...], preferred_element_type=f32)


def _projections(h_or_parts, wniT, wnjT, wsrcT, combine):
    d = wniT.shape[0]
    nb = 10
    bn = N // nb
    hspec = pl.BlockSpec((bn, d), lambda i: (i, 0))
    wspec = pl.BlockSpec((d, HN), lambda i: (0, 0))
    ospec = pl.BlockSpec((bn, HN), lambda i: (i, 0))
    out = jax.ShapeDtypeStruct((N, HN), f32)
    if combine:
        in_specs = [hspec, hspec, wspec, wspec, wspec]
        args = (*h_or_parts, wniT, wnjT, wsrcT)
    else:
        in_specs = [hspec, wspec, wspec, wspec]
        args = (h_or_parts, wniT, wnjT, wsrcT)
    return pl.pallas_call(
        functools.partial(_proj_body, combine=combine),
        grid=(nb,),
        in_specs=in_specs,
        out_specs=[ospec, ospec, ospec],
        out_shape=[out, out, out],
    )(*args)


def _edge_mm_body(f_ref, w_ref, b_ref, o_ref, *, relu):
    f = f_ref[...]
    if relu:
        f = jnp.maximum(f, 0.0)
    o_ref[...] = jnp.dot(f, w_ref[...], preferred_element_type=f32) + b_ref[...]


def _edge_matmul(f, wT, bias, relu):
    d = wT.shape[0]
    nb = 100
    be = E // nb
    return pl.pallas_call(
        functools.partial(_edge_mm_body, relu=relu),
        grid=(nb,),
        in_specs=[pl.BlockSpec((be, d), lambda i: (i, 0)),
                  pl.BlockSpec((d, HE), lambda i: (0, 0)),
                  pl.BlockSpec((1, HE), lambda i: (0, 0))],
        out_specs=pl.BlockSpec((be, HE), lambda i: (i, 0)),
        out_shape=jax.ShapeDtypeStruct((E, HE), f32),
    )(f, wT, bias)


def _mlp_body(f_ref, w0_ref, b0_ref, w1_ref, b1_ref, o_ref):
    f = jnp.maximum(f_ref[...], 0.0)
    x = jnp.dot(f, w0_ref[...], preferred_element_type=f32) + b0_ref[...]
    x = _SELU_S * jnp.where(x > 0.0, x, _SELU_A * (jnp.exp(x) - 1.0))
    o_ref[...] = jnp.sum(x * w1_ref[...], axis=1) + b1_ref[0, 0]


def _final_mlp(fout, w0T, b0, w1, b1):
    nb = 625
    be = E // nb
    dm = w0T.shape[1]
    return pl.pallas_call(
        _mlp_body,
        grid=(nb,),
        in_specs=[pl.BlockSpec((be, HE), lambda i: (i, 0)),
                  pl.BlockSpec((HE, dm), lambda i: (0, 0)),
                  pl.BlockSpec((1, dm), lambda i: (0, 0)),
                  pl.BlockSpec((1, dm), lambda i: (0, 0)),
                  pl.BlockSpec((1, 1), lambda i: (0, 0))],
        out_specs=pl.BlockSpec((be,), lambda i: (i,)),
        out_shape=jax.ShapeDtypeStruct((E,), f32),
    )(fout, w0T, b0, w1, b1)


def _combine_body(p_ref, c_ref, o_ref):
    o_ref[...] = jnp.sum(p_ref[...], axis=0) + c_ref[...]


def _final_combine(parts, charges):
    return pl.pallas_call(
        _combine_body,
        out_shape=jax.ShapeDtypeStruct((N,), f32),
    )(parts, charges)


def _lane():
    return lax.iota(i32, L)


def _any_lanes(pend):
    cnt = plsc.all_reduce_population_count(pend != 0)
    return cnt[0] > 0


def _scatter_update(acc_ref, tag_ref, idx, val, op):
    tslot = lax.bitwise_and(idx, 1023)
    lane = _lane()

    def body(pend_i):
        pend = pend_i != 0
        plsc.store_scatter(tag_ref, [tslot], lane, mask=pend)
        got = plsc.load_gather(tag_ref, [tslot])
        win = jnp.logical_and(got == lane, pend)
        cur = plsc.load_gather(acc_ref, [idx])
        new = op(cur, val)
        plsc.store_scatter(acc_ref, [idx], new, mask=win)
        return jnp.where(win, 0, pend_i)

    lax.while_loop(_any_lanes, body, jnp.ones((L,), i32))


def _fill_1d(ref, n, value):
    def body(i, _):
        ref[pl.ds(i * L, L)] = jnp.full((L,), value, f32)
        return 0
    lax.fori_loop(0, n // L, body, 0)


def _fill_2d(ref, rows, cols, value):
    def body(i, _):
        r = i // (cols // L)
        k = lax.rem(i, cols // L)
        ref[r, pl.ds(k * L, L)] = jnp.full((L,), value, f32)
        return 0
    lax.fori_loop(0, rows * (cols // L), body, 0)


def _m8(x):
    return pl.multiple_of(x, 8)


def _edge_feat_kernel(src2, dst2, pni, pnj, fij, attnb):
    mesh = plsc.VectorSubcoreMesh(core_axis_name="c", subcore_axis_name="s")

    @functools.partial(
        pl.kernel,
        out_type=[jax.ShapeDtypeStruct((E, HE), f32),
                  jax.ShapeDtypeStruct((ROWS2, B), f32)],
        mesh=mesh,
        compiler_params=pltpu.CompilerParams(needs_layout_passes=False, use_tc_tiling_on_sc=False),
        scratch_types=[
            pltpu.VMEM((RW, B), i32),
            pltpu.VMEM((RW, B), i32),
            pltpu.VMEM((B, HE), f32),
            pltpu.VMEM((B, HE), f32),
            pltpu.VMEM((B, HE), f32),
            pltpu.VMEM((B, HE), f32),
            pltpu.VMEM((B, HE), f32),
            pltpu.VMEM((B, HE), f32),
            pltpu.VMEM((B, HE), f32),
            pltpu.VMEM((B,), f32),
            pltpu.VMEM((HE, L), f32),
            pltpu.SemaphoreType.DMA,
            pltpu.SemaphoreType.DMA,
        ],
    )
    def body(src_h, dst_h, pni_h, pnj_h, fij_h, attnb_h, fout_h, e_h,
             src_v, dst_v, ni_0, nj_0, fij_0, ni_1, nj_1, fij_1,
             fo_v, e_v, attn_v, sem0, sem1):
        cid = lax.axis_index("c")
        sid = lax.axis_index("s")
        wid = cid * NS + sid
        row0 = _m8(wid * RW)
        ebase = wid * EW2

        pltpu.sync_copy(src_h.at[pl.ds(row0, RW), :], src_v)
        pltpu.sync_copy(dst_h.at[pl.ds(row0, RW), :], dst_v)
        pltpu.sync_copy(attnb_h, attn_v)

        def _issue(cc, ni_b, nj_b, fi_b, sem):
            e0 = _m8(ebase + cc * B)

            @pl.when(e0 < E)
            def _():
                pltpu.async_copy(pni_h.at[src_v.at[cc]], ni_b, sem)
                pltpu.async_copy(pnj_h.at[dst_v.at[cc]], nj_b, sem)
                pltpu.async_copy(fij_h.at[pl.ds(e0, B), :], fi_b, sem)

        def _wait(cc, ni_b, nj_b, fi_b, sem):
            e0 = _m8(ebase + cc * B)

            @pl.when(e0 < E)
            def _():
                pltpu.make_async_copy(pni_h.at[src_v.at[cc]], ni_b, sem).wait()
                pltpu.make_async_copy(pnj_h.at[dst_v.at[cc]], nj_b, sem).wait()
                pltpu.make_async_copy(fij_h.at[pl.ds(e0, B), :], fi_b, sem).wait()

        def _compute(cc, ni_b, nj_b, fi_b):
            e0 = _m8(ebase + cc * B)

            @pl.when(e0 < E)
            def _():
                def ew(i, _):
                    r = i // 4
                    k = lax.rem(i, 4)
                    sl = pl.ds(k * L, L)
                    x = ni_b[r, sl] + nj_b[r, sl] + fi_b[r, sl]
                    fo_v[r, sl] = jnp.maximum(x, 0.0) + 0.01 * jnp.minimum(x, 0.0)
                    return 0
                lax.fori_loop(0, B * 4, ew, 0)

                def egroup(g, _):
                    rows = _lane() + g * L

                    def edim(d, acc):
                        v = plsc.load_gather(fo_v, [rows, jnp.full((L,), d, i32)])
                        return acc + v * attn_v[d, :]
                    acc = lax.fori_loop(0, HE, edim, jnp.zeros((L,), f32))
                    e_v[pl.ds(g * L, L)] = acc
                    return 0
                lax.fori_loop(0, B // L, egroup, 0)

                pltpu.sync_copy(fo_v, fout_h.at[pl.ds(e0, B), :])
                pltpu.sync_copy(e_v, e_h.at[row0 + cc])

            @pl.when(e0 >= E)
            def _():
                def zf(g, _):
                    e_v[pl.ds(g * L, L)] = jnp.zeros((L,), f32)
                    return 0
                lax.fori_loop(0, B // L, zf, 0)
                pltpu.sync_copy(e_v, e_h.at[row0 + cc])

        _issue(0, ni_0, nj_0, fij_0, sem0)

        def pair(p, _):
            cc0 = 2 * p
            _issue(cc0 + 1, ni_1, nj_1, fij_1, sem1)
            _wait(cc0, ni_0, nj_0, fij_0, sem0)
            _compute(cc0, ni_0, nj_0, fij_0)

            @pl.when(cc0 + 2 < RW)
            def _():
                _issue(cc0 + 2, ni_0, nj_0, fij_0, sem0)

            _wait(cc0 + 1, ni_1, nj_1, fij_1, sem1)
            _compute(cc0 + 1, ni_1, nj_1, fij_1)
            return 0

        lax.fori_loop(0, RW // 2, pair, 0)

    return body(src2, dst2, pni, pnj, fij, attnb)


def _softmax_message_kernel(e2, src2, dst2, psrc):
    mesh = plsc.VectorSubcoreMesh(core_axis_name="c", subcore_axis_name="s")

    @functools.partial(
        pl.kernel,
        out_type=[jax.ShapeDtypeStruct((NC, N, HN), f32),
                  jax.ShapeDtypeStruct((NC, NS, NP), f32),
                  jax.ShapeDtypeStruct((NC, 2, NP), f32)],
        mesh=mesh,
        compiler_params=pltpu.CompilerParams(needs_layout_passes=False, use_tc_tiling_on_sc=False),
        scratch_types=[
            pltpu.VMEM((NP,), f32),
            pltpu.VMEM((NP,), f32),
            pltpu.VMEM((NP,), f32),
            pltpu.VMEM((1024,), i32),
            pltpu.VMEM((SCE // B, B), f32),
            pltpu.VMEM((SCE // B, B), i32),
            pltpu.VMEM((RW, B), i32),
            pltpu.VMEM((RW, B), i32),
            pltpu.VMEM((RW, B), f32),
            pltpu.VMEM((B, HN), f32),
            pltpu.VMEM((B, HN), f32),
            pltpu.VMEM((B,), f32),
            pltpu.VMEM((NT,), f32),
            pltpu.VMEM((NT,), f32),
            pltpu.SemaphoreType.DMA,
            pltpu.SemaphoreType.DMA,
            pltpu.SemaphoreType.DMA,
            pltpu.SemaphoreType.DMA,
            pltpu.VMEM_SHARED((NP, HN), f32),
        ],
    )
    def body(e_h, src_h, dst_h, psrc_h, out_h, stat_h, fin_h,
             acc_loc, m_t, s_t, tag_v, ch_e, ch_d,
             msg_s2, msg_d2, msg_e, rows_0, rows_1, a_v, cmb_a, cmb_i,
             sg0, sg1, sw0, sw1, h_acc):
        cid = lax.axis_index("c")
        sid = lax.axis_index("s")
        wid = cid * NS + sid
        row0 = _m8(wid * RW)
        ebase = wid * EW2

        _fill_1d(acc_loc, NP, -3.0e38)
        _fill_2d(rows_0, B, HN, 0.0)

        def hz(j, _):
            r0 = _m8(sid * NT + j * B)
            pltpu.sync_copy(rows_0, h_acc.at[pl.ds(r0, B), :])
            return 0
        lax.fori_loop(0, NT // B, hz, 0)

        SR = SCE // B

        def max_chunk(cc, _):
            o = _m8(sid * RS + cc * SR)
            pltpu.sync_copy(e_h.at[pl.ds(o, SR), :], ch_e)
            pltpu.sync_copy(dst_h.at[pl.ds(o, SR), :], ch_d)

            def vreg(i, _):
                r = i // (B // L)
                sl = pl.ds(lax.rem(i, B // L) * L, L)
                _scatter_update(acc_loc, tag_v, ch_d[r, sl], ch_e[r, sl],
                                jnp.maximum)
                return 0
            lax.fori_loop(0, SCE // L, vreg, 0)
            return 0
        lax.fori_loop(0, NCH, max_chunk, 0)

        pltpu.sync_copy(acc_loc, stat_h.at[cid, sid])
        plsc.subcore_barrier()

        col0 = _m8(sid * NT)
        pltpu.sync_copy(stat_h.at[cid, 0, pl.ds(col0, NT)], cmb_a)

        def mrow(t, _):
            pltpu.sync_copy(stat_h.at[cid, t, pl.ds(col0, NT)], cmb_i)

            def mv(i, _):
                sl = pl.ds(i * L, L)
                cmb_a[sl] = jnp.maximum(cmb_a[sl], cmb_i[sl])
                return 0
            lax.fori_loop(0, NT // L, mv, 0)
            return 0
        lax.fori_loop(1, NS, mrow, 0)
        pltpu.sync_copy(cmb_a, fin_h.at[cid, 0, pl.ds(col0, NT)])
        plsc.subcore_barrier()
        pltpu.sync_copy(fin_h.at[cid, 0], m_t)

        _fill_1d(acc_loc, NP, 0.0)

        def sum_chunk(cc, _):
            o = _m8(sid * RS + cc * SR)
            pltpu.sync_copy(e_h.at[pl.ds(o, SR), :], ch_e)
            pltpu.sync_copy(dst_h.at[pl.ds(o, SR), :], ch_d)

            def vreg(i, _):
                r = i // (B // L)
                sl = pl.ds(lax.rem(i, B // L) * L, L)
                dv = ch_d[r, sl]
                ex = jnp.exp(ch_e[r, sl] - plsc.load_gather(m_t, [dv]))
                _scatter_update(acc_loc, tag_v, dv, ex, jnp.add)
                return 0
            lax.fori_loop(0, SCE // L, vreg, 0)
            return 0
        lax.fori_loop(0, NCH, sum_chunk, 0)

        pltpu.sync_copy(acc_loc, stat_h.at[cid, sid])
        plsc.subcore_barrier()

        pltpu.sync_copy(stat_h.at[cid, 0, pl.ds(col0, NT)], cmb_a)

        def srow(t, _):
            pltpu.sync_copy(stat_h.at[cid, t, pl.ds(col0, NT)], cmb_i)

            def sv(i, _):
                sl = pl.ds(i * L, L)
                cmb_a[sl] = cmb_a[sl] + cmb_i[sl]
                return 0
            lax.fori_loop(0, NT // L, sv, 0)
            return 0
        lax.fori_loop(1, NS, srow, 0)
        pltpu.sync_copy(cmb_a, fin_h.at[cid, 1, pl.ds(col0, NT)])
        plsc.subcore_barrier()
        pltpu.sync_copy(fin_h.at[cid, 1], s_t)

        pltpu.sync_copy(src_h.at[pl.ds(row0, RW), :], msg_s2)
        pltpu.sync_copy(dst_h.at[pl.ds(row0, RW), :], msg_d2)
        pltpu.sync_copy(e_h.at[pl.ds(row0, RW), :], msg_e)

        def _i3(cc, rb, sem):
            @pl.when(ebase + cc * B < E)
            def _():
                pltpu.async_copy(psrc_h.at[msg_s2.at[cc]], rb, sem)

        def _w3(cc, rb, sem):
            @pl.when(ebase + cc * B < E)
            def _():
                pltpu.make_async_copy(psrc_h.at[msg_s2.at[cc]], rb, sem).wait()

        def _c3(cc, rb):
            @pl.when(ebase + cc * B < E)
            def _():
                def agroup(g, _):
                    sl = pl.ds(g * L, L)
                    dv = msg_d2[cc, sl]
                    ex = jnp.exp(msg_e[cc, sl] - plsc.load_gather(m_t, [dv]))
                    a_v[sl] = ex / plsc.load_gather(s_t, [dv])
                    return 0
                lax.fori_loop(0, B // L, agroup, 0)

                def scale(r, _):
                    ab = plsc.load_gather(a_v, [jnp.full((L,), r, i32)])
                    for k in range(HN // L):
                        sl = pl.ds(k * L, L)
                        rb[r, sl] = rb[r, sl] * ab
                    return 0
                lax.fori_loop(0, B, scale, 0)

        def _s3(cc, rb, sem):
            @pl.when(ebase + cc * B < E)
            def _():
                pltpu.async_copy(rb, h_acc.at[msg_d2.at[cc]], sem, add=True)

        def _sw3(cc, rb, sem):
            @pl.when(ebase + cc * B < E)
            def _():
                pltpu.make_async_copy(rb, h_acc.at[msg_d2.at[cc]], sem).wait()

        _i3(0, rows_0, sg0)

        def pair3(p, _):
            cc0 = 2 * p

            @pl.when(p > 0)
            def _():
                _sw3(cc0 - 1, rows_1, sw1)

            _i3(cc0 + 1, rows_1, sg1)
            _w3(cc0, rows_0, sg0)
            _c3(cc0, rows_0)
            _s3(cc0, rows_0, sw0)

            @pl.when(cc0 + 2 < RW)
            def _():
                _sw3(cc0, rows_0, sw0)
                _i3(cc0 + 2, rows_0, sg0)

            _w3(cc0 + 1, rows_1, sg1)
            _c3(cc0 + 1, rows_1)
            _s3(cc0 + 1, rows_1, sw1)
            return 0

        lax.fori_loop(0, RW // 2, pair3, 0)
        _sw3(RW - 2, rows_0, sw0)
        _sw3(RW - 1, rows_1, sw1)

        plsc.subcore_barrier()

        def hout(j, _):
            r0 = _m8(sid * NT + j * B)

            @pl.when(r0 < N)
            def _():
                pltpu.sync_copy(h_acc.at[pl.ds(r0, B), :],
                                out_h.at[cid, pl.ds(r0, B), :])
            return 0
        lax.fori_loop(0, NT // B, hout, 0)

    return body(e2, src2, dst2, psrc)


def _delta_kernel(inc1, src1, dst1):
    mesh = plsc.VectorSubcoreMesh(core_axis_name="c", subcore_axis_name="s")

    @functools.partial(
        pl.kernel,
        out_type=jax.ShapeDtypeStruct((NW * N,), f32),
        mesh=mesh,
        compiler_params=pltpu.CompilerParams(needs_layout_passes=False, use_tc_tiling_on_sc=False),
        scratch_types=[
            pltpu.VMEM((NP,), f32),
            pltpu.VMEM((1024,), i32),
            pltpu.VMEM((SCE,), f32),
            pltpu.VMEM((SCE,), i32),
            pltpu.VMEM((SCE,), i32),
        ],
    )
    def body(inc_h, src_h, dst_h, out_h, d_loc, tag_v, ch_i, ch_s, ch_d):
        cid = lax.axis_index("c")
        sid = lax.axis_index("s")
        wid = cid * NS + sid
        ebase = wid * EW2

        _fill_1d(d_loc, NP, 0.0)

        def chunk(cc, _):
            o = _m8(ebase + cc * SCE)
            pltpu.sync_copy(inc_h.at[pl.ds(o, SCE)], ch_i)
            pltpu.sync_copy(src_h.at[pl.ds(o, SCE)], ch_s)
            pltpu.sync_copy(dst_h.at[pl.ds(o, SCE)], ch_d)

            def vreg(i, _):
                sl = pl.ds(i * L, L)
                iv = ch_i[sl]
                _scatter_update(d_loc, tag_v, ch_d[sl], iv, jnp.add)
                _scatter_update(d_loc, tag_v, ch_s[sl], -iv, jnp.add)
                return 0
            lax.fori_loop(0, SCE // L, vreg, 0)
            return 0
        lax.fori_loop(0, EW2 // SCE, chunk, 0)

        pltpu.sync_copy(d_loc.at[pl.ds(0, N)], out_h.at[pl.ds(_m8(wid * N), N)])

    return body(inc1, src1, dst1)


def kernel(edge_index, feats_node, feats_edge, charges_init,
           w0_src, w0_ni, w0_nj, w0_fij, a0, b0,
           w1_src, w1_ni, w1_nj, w1_fij, a1, b1,
           w2_src, w2_ni, w2_nj, w2_fij, a2, b2,
           mlp_w0, mlp_b0, mlp_w1, mlp_b1):
    src = edge_index[0]
    dst = edge_index[1]
    pad0 = jnp.zeros((EPAD,), i32)
    dummy = (N + (jnp.arange(EPAD, dtype=i32) % (NP - N))).astype(i32)
    src2 = jnp.concatenate([src, pad0]).reshape(ROWS2, B)
    src1 = jnp.concatenate([src, dummy])
    dst1 = jnp.concatenate([dst, dummy])
    dst2 = dst1.reshape(ROWS2, B)

    layers = [(w0_src, w0_ni, w0_nj, w0_fij, a0, b0),
              (w1_src, w1_ni, w1_nj, w1_fij, a1, b1),
              (w2_src, w2_ni, w2_nj, w2_fij, a2, b2)]

    h_parts = None
    fout = None
    for li, (Ws, Wni, Wnj, Wf, at, bi) in enumerate(layers):
        attnb = jnp.broadcast_to(at.reshape(HE, 1), (HE, L))
        if li == 0:
            pni, pnj, psrc = _projections(feats_node, Wni.T, Wnj.T, Ws.T,
                                          combine=False)
            fij = _edge_matmul(feats_edge, Wf.T, bi.reshape(1, HE), relu=False)
        else:
            pni, pnj, psrc = _projections(h_parts, Wni.T, Wnj.T, Ws.T,
                                          combine=True)
            fij = _edge_matmul(fout, Wf.T, bi.reshape(1, HE), relu=True)
        fout, e2 = _edge_feat_kernel(src2, dst2, pni, pnj, fij, attnb)
        hp, _, _ = _softmax_message_kernel(e2, src2, dst2, psrc)
        h_parts = (hp[0], hp[1])

    inc = _final_mlp(fout, mlp_w0.T, mlp_b0.reshape(1, -1),
                     mlp_w1.reshape(1, -1), mlp_b1.reshape(1, 1))
    inc1 = jnp.concatenate([inc, jnp.zeros((EPAD,), f32)])
    parts = _delta_kernel(inc1, src1, dst1)
    return _final_combine(parts.reshape(NW, N), charges_init)
